# Initial kernel scaffold; baseline (speedup 1.0000x reference)
import functools

import jax
import jax.numpy as jnp
from jax import lax
from jax.experimental import pallas as pl
from jax.experimental.pallas import tpu as pltpu
from jax.experimental.pallas import tpu_sc as plsc

N = 10000
E = 160000
D = 256
HALF = 128

NC = 2
NS = 16
CHUNK = 128
EPAD = 163840
EPT = EPAD // NS
CPT = EPT // CHUNK
NROW = N + 16
RPT = NROW // NS
OPT = N // NS
BN = 1000

_mesh = plsc.VectorSubcoreMesh(core_axis_name="c", subcore_axis_name="s")


@functools.partial(
    pl.kernel,
    out_type=jax.ShapeDtypeStruct((NC, N, 16), jnp.float32),
    mesh=_mesh,
    scratch_types=[
        pltpu.VMEM((CPT, CHUNK), jnp.int32),
        pltpu.VMEM((CHUNK, 16), jnp.float32),
        pltpu.VMEM((RPT, 16), jnp.float32),
        pltpu.VMEM_SHARED((NROW, 16), jnp.float32),
    ],
)
def _deg_kernel(degidx_hbm, out_hbm, idx_v, ones_v, zb_v, acc):
    c = lax.axis_index("c")
    s = lax.axis_index("s")
    pltpu.sync_copy(degidx_hbm.at[c, s], idx_v)

    def _fill(i, _):
        ones_v[i, :] = jnp.ones((16,), jnp.float32)
        return 0

    lax.fori_loop(0, CHUNK, _fill, 0)

    def _zero(i, _):
        zb_v[i, :] = jnp.zeros((16,), jnp.float32)
        return 0

    lax.fori_loop(0, RPT, _zero, 0)
    pltpu.sync_copy(zb_v, acc.at[pl.ds(s * RPT, RPT)])
    plsc.subcore_barrier()
    for j in range(CPT):
        pltpu.sync_copy(ones_v, acc.at[idx_v.at[j]], add=True)
    plsc.subcore_barrier()
    pltpu.sync_copy(acc.at[pl.ds(s * OPT, OPT)], out_hbm.at[c, pl.ds(s * OPT, OPT)])


@functools.partial(
    pl.kernel,
    out_type=jax.ShapeDtypeStruct((NC, N, HALF), jnp.float32),
    mesh=_mesh,
    scratch_types=[
        pltpu.VMEM((EPT,), jnp.int32),
        pltpu.VMEM((CPT, CHUNK), jnp.int32),
        pltpu.VMEM((CHUNK, HALF), jnp.float32),
        pltpu.VMEM((CHUNK, HALF), jnp.float32),
        pltpu.VMEM((64, HALF), jnp.float32),
        pltpu.VMEM_SHARED((NROW, HALF), jnp.float32),
        pltpu.SemaphoreType.DMA,
        pltpu.SemaphoreType.DMA,
    ],
)
def _agg_kernel(y_hbm, srcw_hbm, dstw_hbm, out_hbm, src_v, dst_v, rows_a,
                rows_b, zb_v, acc, sem_a, sem_b):
    c = lax.axis_index("c")
    s = lax.axis_index("s")
    pltpu.sync_copy(srcw_hbm.at[c, s], src_v)
    pltpu.sync_copy(dstw_hbm.at[s], dst_v)

    def _zero(i, _):
        for j in range(HALF // 16):
            zb_v[i, pl.ds(j * 16, 16)] = jnp.zeros((16,), jnp.float32)
        return 0

    lax.fori_loop(0, 64, _zero, 0)
    base = s * RPT
    for k in range(RPT // 64):
        pltpu.sync_copy(zb_v, acc.at[pl.ds(base + k * 64, 64)])
    rem = RPT % 64
    if rem:
        pltpu.sync_copy(zb_v.at[pl.ds(0, rem)],
                        acc.at[pl.ds(base + (RPT // 64) * 64, rem)])
    plsc.subcore_barrier()

    rows = (rows_a, rows_b)
    sems = (sem_a, sem_b)
    pending = pltpu.async_copy(y_hbm.at[src_v.at[pl.ds(0, CHUNK)]], rows[0],
                               sems[0])
    for j in range(CPT):
        nxt = None
        if j + 1 < CPT:
            nxt = pltpu.async_copy(
                y_hbm.at[src_v.at[pl.ds((j + 1) * CHUNK, CHUNK)]],
                rows[(j + 1) % 2], sems[(j + 1) % 2])
        pending.wait()
        pltpu.sync_copy(rows[j % 2], acc.at[dst_v.at[j]], add=True)
        pending = nxt
    plsc.subcore_barrier()
    pltpu.sync_copy(acc.at[pl.ds(s * OPT, OPT)], out_hbm.at[c, pl.ds(s * OPT, OPT)])


def _tc1_body(degs_ref, degd_ref, x_ref, w_ref, y_ref, ns_ref, nd_ref):
    ns = lax.rsqrt(jnp.clip(degs_ref[0, :, 0:1], 1.0, None))
    nd = lax.rsqrt(jnp.clip(degd_ref[0, :, 0:1], 1.0, None))
    h = x_ref[...] * ns
    y_ref[...] = jnp.dot(h, w_ref[...], precision=lax.Precision.HIGHEST,
                         preferred_element_type=jnp.float32)
    ns_ref[...] = ns
    nd_ref[...] = nd


def _tc1(degs, x, w1):
    nb = N // BN
    return pl.pallas_call(
        _tc1_body,
        grid=(NC, nb),
        in_specs=[
            pl.BlockSpec((1, BN, 16), lambda c, i: (0, i, 0)),
            pl.BlockSpec((1, BN, 16), lambda c, i: (1, i, 0)),
            pl.BlockSpec((BN, D), lambda c, i: (i, 0)),
            pl.BlockSpec((D, HALF), lambda c, i: (0, c)),
        ],
        out_specs=[
            pl.BlockSpec((BN, HALF), lambda c, i: (c * (N // BN) + i, 0)),
            pl.BlockSpec((BN, 1), lambda c, i: (i, 0)),
            pl.BlockSpec((BN, 1), lambda c, i: (i, 0)),
        ],
        out_shape=[
            jax.ShapeDtypeStruct((2 * N, HALF), jnp.float32),
            jax.ShapeDtypeStruct((N, 1), jnp.float32),
            jax.ShapeDtypeStruct((N, 1), jnp.float32),
        ],
    )(degs, degs, x, w1)


def _tcmid_body(a0_ref, a1_ref, ns_ref, nd_ref, b_ref, w_ref, y_ref):
    h = jnp.concatenate([a0_ref[0], a1_ref[0]], axis=1)
    h = jax.nn.relu(h * nd_ref[...] + b_ref[...][None, :])
    y_ref[...] = jnp.dot(h * ns_ref[...], w_ref[...],
                         precision=lax.Precision.HIGHEST,
                         preferred_element_type=jnp.float32)


def _tcmid(agg, ns, nd, b, w):
    return pl.pallas_call(
        _tcmid_body,
        grid=(NC, N // BN),
        in_specs=[
            pl.BlockSpec((1, BN, HALF), lambda c, i: (0, i, 0)),
            pl.BlockSpec((1, BN, HALF), lambda c, i: (1, i, 0)),
            pl.BlockSpec((BN, 1), lambda c, i: (i, 0)),
            pl.BlockSpec((BN, 1), lambda c, i: (i, 0)),
            pl.BlockSpec((D,), lambda c, i: (0,)),
            pl.BlockSpec((D, HALF), lambda c, i: (0, c)),
        ],
        out_specs=pl.BlockSpec((BN, HALF), lambda c, i: (c * (N // BN) + i, 0)),
        out_shape=jax.ShapeDtypeStruct((2 * N, HALF), jnp.float32),
    )(agg, agg, ns, nd, b, w)


def _tcfin_body(a_ref, nd_ref, b_ref, o_ref):
    o_ref[...] = jax.nn.relu(a_ref[0] * nd_ref[...] + b_ref[...][None, :])


def _tcfin(agg, nd, b):
    return pl.pallas_call(
        _tcfin_body,
        grid=(NC, N // BN),
        in_specs=[
            pl.BlockSpec((1, BN, HALF), lambda c, i: (c, i, 0)),
            pl.BlockSpec((BN, 1), lambda c, i: (i, 0)),
            pl.BlockSpec((HALF,), lambda c, i: (c,)),
        ],
        out_specs=pl.BlockSpec((BN, HALF), lambda c, i: (i, c)),
        out_shape=jax.ShapeDtypeStruct((N, D), jnp.float32),
    )(agg, nd, b)


def kernel(x, edge_index, W1, b1, W2, b2, W3, b3):
    src = edge_index[0].astype(jnp.int32)
    dst = edge_index[1].astype(jnp.int32)
    pad = EPAD - E
    padN = jnp.full((pad,), N, jnp.int32)
    src_g = jnp.concatenate([src, jnp.zeros((pad,), jnp.int32)])
    dst_p = jnp.concatenate([dst, padN])
    src_d = jnp.concatenate([src, padN])

    degidx = jnp.stack([src_d.reshape(NS, CPT, CHUNK),
                        dst_p.reshape(NS, CPT, CHUNK)])
    srcr = src_g.reshape(NS, EPT)
    srcw = jnp.stack([srcr, srcr + N])
    dstw = dst_p.reshape(NS, CPT, CHUNK)

    degs = _deg_kernel(degidx)
    y1, ns, nd = _tc1(degs, x, W1)
    agg1 = _agg_kernel(y1, srcw, dstw)
    y2 = _tcmid(agg1, ns, nd, b1, W2)
    agg2 = _agg_kernel(y2, srcw, dstw)
    y3 = _tcmid(agg2, ns, nd, b2, W3)
    agg3 = _agg_kernel(y3, srcw, dstw)
    return _tcfin(agg3, nd, b3)


# speedup vs baseline: 1.6252x; 1.6252x over previous
import functools

import jax
import jax.numpy as jnp
from jax import lax
from jax.experimental import pallas as pl
from jax.experimental.pallas import tpu as pltpu
from jax.experimental.pallas import tpu_sc as plsc

N = 10000
E = 160000
D = 256
HALF = 128

NC = 2
NS = 16
NP = 2
NH = N // NP
CHUNK = 128
EPAD = 163840
EPT = EPAD // NS
CPT = EPT // CHUNK
NRD = 16000
RPTD = NRD // NS
NROWH = NH + 24
RPTH = NROWH // NS
OPTH = 312
TAILH = NH - NS * OPTH
BN = 1000


def _deg_body(degidx_hbm, ones_hbm, zer_hbm, out_hbm, idx_v, ones_v, buf_v,
              acc):
    c = lax.axis_index("c")
    s = lax.axis_index("s")
    w = c * NS + s
    pltpu.sync_copy(degidx_hbm.at[w], idx_v)
    pltpu.sync_copy(ones_hbm, ones_v)
    pltpu.sync_copy(zer_hbm.at[pl.ds(s * RPTD, RPTD)], buf_v)
    pltpu.sync_copy(buf_v, acc.at[pl.ds(s * RPTD, RPTD)])
    plsc.subcore_barrier()

    def _scat(j, _):
        pltpu.sync_copy(ones_v, acc.at[idx_v.at[j]], add=True)
        return 0

    lax.fori_loop(0, CPT, _scat, 0)
    plsc.subcore_barrier()
    pltpu.sync_copy(acc.at[pl.ds(s * RPTD, RPTD)], buf_v)
    pltpu.sync_copy(buf_v, out_hbm.at[pl.ds(c * NRD + s * RPTD, RPTD)])


def _agg_body(y_hbm, srcw_hbm, dstp_hbm, zer_hbm, out_hbm, src_v, dst_v,
              rows_a, rows_b, acc, sem):
    c = lax.axis_index("c")
    s = lax.axis_index("s")
    w = c * NS + s
    pltpu.sync_copy(srcw_hbm.at[w], src_v)
    rows = (rows_a, rows_b)
    U = len(rows)

    for p in range(NP):
        pltpu.sync_copy(dstp_hbm.at[p * NS + s], dst_v)
        pltpu.sync_copy(zer_hbm.at[s], acc.at[pl.ds(s * RPTH, RPTH)])
        plsc.subcore_barrier()

        def _grp(g, _):
            cps = [pltpu.async_copy(y_hbm.at[src_v.at[g * U + b]], rows[b],
                                    sem) for b in range(U)]
            for b in range(U):
                cps[b].wait()
            for b in range(U):
                pltpu.sync_copy(rows[b], acc.at[dst_v.at[g * U + b]], add=True)
            return 0

        lax.fori_loop(0, CPT // U, _grp, 0)
        plsc.subcore_barrier()
        pltpu.sync_copy(acc.at[pl.ds(s * RPTH, RPTH)],
                        out_hbm.at[(c * NP + p) * NS + s])
        plsc.subcore_barrier()


@functools.cache
def _sc_kernels():
    mesh = plsc.VectorSubcoreMesh(core_axis_name="c", subcore_axis_name="s",
                                  num_cores=NC, num_subcores=NS)
    deg = pl.kernel(
        _deg_body,
        out_type=jax.ShapeDtypeStruct((2 * NRD,), jnp.float32),
        mesh=mesh,
        scratch_types=[
            pltpu.VMEM((CPT, CHUNK), jnp.int32),
            pltpu.VMEM((CHUNK,), jnp.float32),
            pltpu.VMEM((RPTD,), jnp.float32),
            pltpu.VMEM_SHARED((NRD,), jnp.float32),
        ],
    )
    agg = pl.kernel(
        _agg_body,
        out_type=jax.ShapeDtypeStruct((NC * NP * NS, RPTH, HALF), jnp.float32),
        mesh=mesh,
        scratch_types=[
            pltpu.VMEM((CPT, CHUNK), jnp.int32),
            pltpu.VMEM((CPT, CHUNK), jnp.int32),
            pltpu.VMEM((CHUNK, HALF), jnp.float32),
            pltpu.VMEM((CHUNK, HALF), jnp.float32),
            pltpu.VMEM_SHARED((NROWH, HALF), jnp.float32),
            pltpu.SemaphoreType.DMA,
        ],
    )
    return deg, agg


def _tcnorm_body(d_ref, ns_ref, nd_ref):
    a = lax.rsqrt(jnp.clip(d_ref[...], 1.0, None))
    ns_ref[...] = a[0:N].reshape(N, 1)
    nd_ref[...] = a[NRD:NRD + N].reshape(N, 1)


def _tcnorm(degs):
    return pl.pallas_call(
        _tcnorm_body,
        out_shape=[
            jax.ShapeDtypeStruct((N, 1), jnp.float32),
            jax.ShapeDtypeStruct((N, 1), jnp.float32),
        ],
    )(degs)


def _tc1_body(ns_ref, x_ref, w_ref, y_ref):
    h = x_ref[...] * ns_ref[...]
    y_ref[...] = jnp.dot(h, w_ref[...], precision=lax.Precision.HIGHEST,
                         preferred_element_type=jnp.float32)


def _tc1(ns, x, w1):
    nb = N // BN
    return pl.pallas_call(
        _tc1_body,
        grid=(NC, nb),
        in_specs=[
            pl.BlockSpec((BN, 1), lambda c, i: (i, 0)),
            pl.BlockSpec((BN, D), lambda c, i: (i, 0)),
            pl.BlockSpec((D, HALF), lambda c, i: (0, c)),
        ],
        out_specs=pl.BlockSpec((BN, HALF), lambda c, i: (c * nb + i, 0)),
        out_shape=jax.ShapeDtypeStruct((NC * N, HALF), jnp.float32),
    )(ns, x, w1)


def _tcmid_body(a0, a1, ns_ref, nd_ref, b_ref, w_ref, y_ref):
    h = jnp.concatenate([a0[0], a1[0]], axis=1)
    h = jax.nn.relu(h * nd_ref[...] + b_ref[...][None, :])
    y_ref[...] = jnp.dot(h * ns_ref[...], w_ref[...],
                         precision=lax.Precision.HIGHEST,
                         preferred_element_type=jnp.float32)


def _tcmid(agg, ns, nd, b, w):
    return pl.pallas_call(
        _tcmid_body,
        grid=(NC, N // BN),
        in_specs=[
            pl.BlockSpec((1, BN, HALF), lambda c, i: (0, i, 0)),
            pl.BlockSpec((1, BN, HALF), lambda c, i: (1, i, 0)),
            pl.BlockSpec((BN, 1), lambda c, i: (i, 0)),
            pl.BlockSpec((BN, 1), lambda c, i: (i, 0)),
            pl.BlockSpec((D,), lambda c, i: (0,)),
            pl.BlockSpec((D, HALF), lambda c, i: (0, c)),
        ],
        out_specs=pl.BlockSpec((BN, HALF), lambda c, i: (c * (N // BN) + i, 0)),
        out_shape=jax.ShapeDtypeStruct((NC * N, HALF), jnp.float32),
    )(agg, agg, ns, nd, b, w)


def _tcfin_body(a0, a1, nd_ref, b_ref, o_ref):
    h = jnp.concatenate([a0[0], a1[0]], axis=1)
    o_ref[...] = jax.nn.relu(h * nd_ref[...] + b_ref[...][None, :])


def _tcfin(agg, nd, b):
    return pl.pallas_call(
        _tcfin_body,
        grid=(N // BN,),
        in_specs=[
            pl.BlockSpec((1, BN, HALF), lambda i: (0, i, 0)),
            pl.BlockSpec((1, BN, HALF), lambda i: (1, i, 0)),
            pl.BlockSpec((BN, 1), lambda i: (i, 0)),
            pl.BlockSpec((D,), lambda i: (0,)),
        ],
        out_specs=pl.BlockSpec((BN, D), lambda i: (i, 0)),
        out_shape=jax.ShapeDtypeStruct((N, D), jnp.float32),
    )(agg, agg, nd, b)


def kernel(x, edge_index, W1, b1, W2, b2, W3, b3):
    src = edge_index[0].astype(jnp.int32)
    dst = edge_index[1].astype(jnp.int32)
    pad = EPAD - E
    padN = jnp.full((pad,), N, jnp.int32)
    src_g = jnp.concatenate([src, jnp.zeros((pad,), jnp.int32)])
    dst_p = jnp.concatenate([dst, padN])
    src_d = jnp.concatenate([src, padN])

    degidx = jnp.concatenate([src_d.reshape(NS, CPT, CHUNK),
                              dst_p.reshape(NS, CPT, CHUNK)])
    srcr = src_g.reshape(NS, CPT, CHUNK)
    srcw = jnp.concatenate([srcr, srcr + N])
    dstp = jnp.concatenate([
        jnp.where((dst_p >= p * NH) & (dst_p < (p + 1) * NH),
                  dst_p - p * NH, NH).reshape(NS, CPT, CHUNK)
        for p in range(NP)
    ])
    zer_agg = jnp.zeros((NS, RPTH, HALF), jnp.float32)

    deg_k, agg_k = _sc_kernels()

    def agg_kernel(y):
        flat = agg_k(y, srcw, dstp, zer_agg)
        a = flat.reshape(NC, NP, NS * RPTH, HALF)[:, :, :NH, :]
        return a.reshape(NC, N, HALF)

    degs = deg_k(degidx, jnp.ones((CHUNK,), jnp.float32),
                 jnp.zeros((NRD,), jnp.float32))
    ns, nd = _tcnorm(degs)
    y1 = _tc1(ns, x, W1)
    agg1 = agg_kernel(y1)
    y2 = _tcmid(agg1, ns, nd, b1, W2)
    agg2 = agg_kernel(y2)
    y3 = _tcmid(agg2, ns, nd, b2, W3)
    agg3 = agg_kernel(y3)
    return _tcfin(agg3, nd, b3)


# speedup vs baseline: 1.8463x; 1.1360x over previous
import functools

import jax
import jax.numpy as jnp
from jax import lax
from jax.experimental import pallas as pl
from jax.experimental.pallas import tpu as pltpu
from jax.experimental.pallas import tpu_sc as plsc

N = 10000
E = 160000
D = 256
HALF = 128

NC = 2
NS = 16
NP = 2
NH = N // NP
CHUNK = 128
EPAD = 163840
EPT = EPAD // NS
CPT = EPT // CHUNK
NRD = 16000
RPTD = NRD // NS
NROWH = NH + 24
RPTH = NROWH // NS
OPTH = 312
TAILH = NH - NS * OPTH
BN = 1000


def _deg_body(degidx_hbm, ones_hbm, zer_hbm, out_hbm, idx_v, ones_v, buf_v,
              acc):
    c = lax.axis_index("c")
    s = lax.axis_index("s")
    w = c * NS + s
    pltpu.sync_copy(degidx_hbm.at[w], idx_v)
    pltpu.sync_copy(ones_hbm, ones_v)
    pltpu.sync_copy(zer_hbm.at[pl.ds(s * RPTD, RPTD)], buf_v)
    pltpu.sync_copy(buf_v, acc.at[pl.ds(s * RPTD, RPTD)])
    plsc.subcore_barrier()

    def _scat(j, _):
        pltpu.sync_copy(ones_v, acc.at[idx_v.at[j]], add=True)
        return 0

    lax.fori_loop(0, CPT, _scat, 0)
    plsc.subcore_barrier()
    pltpu.sync_copy(acc.at[pl.ds(s * RPTD, RPTD)], buf_v)
    pltpu.sync_copy(buf_v, out_hbm.at[pl.ds(c * NRD + s * RPTD, RPTD)])


def _agg_body(y_hbm, srcw_hbm, dstp_hbm, zer_hbm, out_hbm, src_v, dst_v,
              rows_a, rows_b, rows_c, rows_d, acc, sem_a, sem_b):
    c = lax.axis_index("c")
    s = lax.axis_index("s")
    w = c * NS + s
    pltpu.sync_copy(srcw_hbm.at[w], src_v)
    bufs = ((rows_a, rows_b), (rows_c, rows_d))
    sems = (sem_a, sem_b)
    U = 2
    NG = CPT // U

    def _fire(g, par):
        for b in range(U):
            pltpu.async_copy(y_hbm.at[src_v.at[g * U + b]], bufs[par][b],
                             sems[par])

    def _drain_scatter(g, par):
        for b in range(U):
            pltpu.make_async_copy(y_hbm.at[src_v.at[g * U + b]],
                                  bufs[par][b], sems[par]).wait()
        for b in range(U):
            pltpu.sync_copy(bufs[par][b], acc.at[dst_v.at[g * U + b]],
                            add=True)

    for p in range(NP):
        pltpu.sync_copy(dstp_hbm.at[p * NS + s], dst_v)
        pltpu.sync_copy(zer_hbm.at[s], acc.at[pl.ds(s * RPTH, RPTH)])
        plsc.subcore_barrier()

        _fire(0, 0)

        def _grp2(h, _):
            g0 = 2 * h
            _fire(g0 + 1, 1)
            _drain_scatter(g0, 0)
            _fire(g0 + 2, 0)
            _drain_scatter(g0 + 1, 1)
            return 0

        lax.fori_loop(0, NG // 2 - 1, _grp2, 0)
        _fire(NG - 1, 1)
        _drain_scatter(NG - 2, 0)
        _drain_scatter(NG - 1, 1)
        plsc.subcore_barrier()
        pltpu.sync_copy(acc.at[pl.ds(s * RPTH, RPTH)],
                        out_hbm.at[(c * NP + p) * NS + s])
        plsc.subcore_barrier()


@functools.cache
def _sc_kernels():
    mesh = plsc.VectorSubcoreMesh(core_axis_name="c", subcore_axis_name="s",
                                  num_cores=NC, num_subcores=NS)
    deg = pl.kernel(
        _deg_body,
        out_type=jax.ShapeDtypeStruct((2 * NRD,), jnp.float32),
        mesh=mesh,
        scratch_types=[
            pltpu.VMEM((CPT, CHUNK), jnp.int32),
            pltpu.VMEM((CHUNK,), jnp.float32),
            pltpu.VMEM((RPTD,), jnp.float32),
            pltpu.VMEM_SHARED((NRD,), jnp.float32),
        ],
    )
    agg = pl.kernel(
        _agg_body,
        out_type=jax.ShapeDtypeStruct((NC * NP * NS, RPTH, HALF), jnp.float32),
        mesh=mesh,
        scratch_types=[
            pltpu.VMEM((CPT, CHUNK), jnp.int32),
            pltpu.VMEM((CPT, CHUNK), jnp.int32),
            pltpu.VMEM((CHUNK, HALF), jnp.float32),
            pltpu.VMEM((CHUNK, HALF), jnp.float32),
            pltpu.VMEM((CHUNK, HALF), jnp.float32),
            pltpu.VMEM((CHUNK, HALF), jnp.float32),
            pltpu.VMEM_SHARED((NROWH, HALF), jnp.float32),
            pltpu.SemaphoreType.DMA,
            pltpu.SemaphoreType.DMA,
        ],
    )
    return deg, agg


def _tcnorm_body(d_ref, ns_ref, nd_ref):
    a = lax.rsqrt(jnp.clip(d_ref[...], 1.0, None))
    ns_ref[...] = a[0:N].reshape(N, 1)
    nd_ref[...] = a[NRD:NRD + N].reshape(N, 1)


def _tcnorm(degs):
    return pl.pallas_call(
        _tcnorm_body,
        out_shape=[
            jax.ShapeDtypeStruct((N, 1), jnp.float32),
            jax.ShapeDtypeStruct((N, 1), jnp.float32),
        ],
    )(degs)


def _tc1_body(ns_ref, x_ref, w_ref, y_ref):
    h = x_ref[...] * ns_ref[...]
    y_ref[...] = jnp.dot(h, w_ref[...], precision=lax.Precision.HIGHEST,
                         preferred_element_type=jnp.float32)


def _tc1(ns, x, w1):
    nb = N // BN
    return pl.pallas_call(
        _tc1_body,
        grid=(NC, nb),
        in_specs=[
            pl.BlockSpec((BN, 1), lambda c, i: (i, 0)),
            pl.BlockSpec((BN, D), lambda c, i: (i, 0)),
            pl.BlockSpec((D, HALF), lambda c, i: (0, c)),
        ],
        out_specs=pl.BlockSpec((BN, HALF), lambda c, i: (c * nb + i, 0)),
        out_shape=jax.ShapeDtypeStruct((NC * N, HALF), jnp.float32),
    )(ns, x, w1)


def _tcmid_body(a0, a1, ns_ref, nd_ref, b_ref, w_ref, y_ref):
    h = jnp.concatenate([a0[0], a1[0]], axis=1)
    h = jax.nn.relu(h * nd_ref[...] + b_ref[...][None, :])
    y_ref[...] = jnp.dot(h * ns_ref[...], w_ref[...],
                         precision=lax.Precision.HIGHEST,
                         preferred_element_type=jnp.float32)


def _tcmid(agg, ns, nd, b, w):
    return pl.pallas_call(
        _tcmid_body,
        grid=(NC, N // BN),
        in_specs=[
            pl.BlockSpec((1, BN, HALF), lambda c, i: (0, i, 0)),
            pl.BlockSpec((1, BN, HALF), lambda c, i: (1, i, 0)),
            pl.BlockSpec((BN, 1), lambda c, i: (i, 0)),
            pl.BlockSpec((BN, 1), lambda c, i: (i, 0)),
            pl.BlockSpec((D,), lambda c, i: (0,)),
            pl.BlockSpec((D, HALF), lambda c, i: (0, c)),
        ],
        out_specs=pl.BlockSpec((BN, HALF), lambda c, i: (c * (N // BN) + i, 0)),
        out_shape=jax.ShapeDtypeStruct((NC * N, HALF), jnp.float32),
    )(agg, agg, ns, nd, b, w)


def _tcfin_body(a0, a1, nd_ref, b_ref, o_ref):
    h = jnp.concatenate([a0[0], a1[0]], axis=1)
    o_ref[...] = jax.nn.relu(h * nd_ref[...] + b_ref[...][None, :])


def _tcfin(agg, nd, b):
    return pl.pallas_call(
        _tcfin_body,
        grid=(N // BN,),
        in_specs=[
            pl.BlockSpec((1, BN, HALF), lambda i: (0, i, 0)),
            pl.BlockSpec((1, BN, HALF), lambda i: (1, i, 0)),
            pl.BlockSpec((BN, 1), lambda i: (i, 0)),
            pl.BlockSpec((D,), lambda i: (0,)),
        ],
        out_specs=pl.BlockSpec((BN, D), lambda i: (i, 0)),
        out_shape=jax.ShapeDtypeStruct((N, D), jnp.float32),
    )(agg, agg, nd, b)


def kernel(x, edge_index, W1, b1, W2, b2, W3, b3):
    src = edge_index[0].astype(jnp.int32)
    dst = edge_index[1].astype(jnp.int32)
    pad = EPAD - E
    padN = jnp.full((pad,), N, jnp.int32)
    src_g = jnp.concatenate([src, jnp.zeros((pad,), jnp.int32)])
    dst_p = jnp.concatenate([dst, padN])
    src_d = jnp.concatenate([src, padN])

    degidx = jnp.concatenate([src_d.reshape(NS, CPT, CHUNK),
                              dst_p.reshape(NS, CPT, CHUNK)])
    srcr = src_g.reshape(NS, CPT, CHUNK)
    srcw = jnp.concatenate([srcr, srcr + N])
    dstp = jnp.concatenate([
        jnp.where((dst_p >= p * NH) & (dst_p < (p + 1) * NH),
                  dst_p - p * NH, NH).reshape(NS, CPT, CHUNK)
        for p in range(NP)
    ])
    zer_agg = jnp.zeros((NS, RPTH, HALF), jnp.float32)

    deg_k, agg_k = _sc_kernels()

    def agg_kernel(y):
        flat = agg_k(y, srcw, dstp, zer_agg)
        a = flat.reshape(NC, NP, NS * RPTH, HALF)[:, :, :NH, :]
        return a.reshape(NC, N, HALF)

    degs = deg_k(degidx, jnp.ones((CHUNK,), jnp.float32),
                 jnp.zeros((NRD,), jnp.float32))
    ns, nd = _tcnorm(degs)
    y1 = _tc1(ns, x, W1)
    agg1 = agg_kernel(y1)
    y2 = _tcmid(agg1, ns, nd, b1, W2)
    agg2 = agg_kernel(y2)
    y3 = _tcmid(agg2, ns, nd, b2, W3)
    agg3 = agg_kernel(y3)
    return _tcfin(agg3, nd, b3)


# speedup vs baseline: 1.8474x; 1.0006x over previous
import functools

import jax
import jax.numpy as jnp
from jax import lax
from jax.experimental import pallas as pl
from jax.experimental.pallas import tpu as pltpu
from jax.experimental.pallas import tpu_sc as plsc

N = 10000
E = 160000
D = 256
HALF = 128

NC = 2
NS = 16
NP = 2
NH = N // NP
CHUNK = 128
EPAD = 163840
EPT = EPAD // NS
CPT = EPT // CHUNK
NRD = 16000
RPTD = NRD // NS
NROWH = NH + 24
RPTH = NROWH // NS
BN = 1000


def _deg_body(degidx_hbm, ones_hbm, zer_hbm, out_hbm, idx_v, ones_v, buf_v,
              acc):
    c = lax.axis_index("c")
    s = lax.axis_index("s")
    w = c * NS + s
    pltpu.sync_copy(degidx_hbm.at[w], idx_v)
    pltpu.sync_copy(ones_hbm, ones_v)
    pltpu.sync_copy(zer_hbm.at[pl.ds(s * RPTD, RPTD)], buf_v)
    pltpu.sync_copy(buf_v, acc.at[pl.ds(s * RPTD, RPTD)])
    plsc.subcore_barrier()

    def _scat(j, _):
        pltpu.sync_copy(ones_v, acc.at[idx_v.at[j]], add=True)
        return 0

    lax.fori_loop(0, CPT, _scat, 0)
    plsc.subcore_barrier()
    pltpu.sync_copy(acc.at[pl.ds(s * RPTD, RPTD)], buf_v)
    pltpu.sync_copy(buf_v, out_hbm.at[pl.ds(c * NRD + s * RPTD, RPTD)])


def _agg_body(y_hbm, srcw_hbm, dstp_hbm, zer_hbm, out_hbm, src_v, dst_v,
              rows_a, rows_b, rows_c, rows_d, acc, sem_a, sem_b):
    c = lax.axis_index("c")
    s = lax.axis_index("s")
    w = c * NS + s
    pltpu.sync_copy(srcw_hbm.at[w], src_v)
    bufs = ((rows_a, rows_b), (rows_c, rows_d))
    sems = (sem_a, sem_b)
    U = 2
    NG = CPT // U

    def _fire(g, par):
        for b in range(U):
            pltpu.async_copy(y_hbm.at[src_v.at[g * U + b]], bufs[par][b],
                             sems[par])

    def _drain_scatter(g, par):
        for b in range(U):
            pltpu.make_async_copy(y_hbm.at[src_v.at[g * U + b]],
                                  bufs[par][b], sems[par]).wait()
        for b in range(U):
            pltpu.sync_copy(bufs[par][b], acc.at[dst_v.at[g * U + b]],
                            add=True)

    for p in range(NP):
        pltpu.sync_copy(dstp_hbm.at[p * NS + s], dst_v)
        pltpu.sync_copy(zer_hbm.at[s], acc.at[pl.ds(s * RPTH, RPTH)])
        plsc.subcore_barrier()

        _fire(0, 0)

        def _grp2(h, _):
            g0 = 2 * h
            _fire(g0 + 1, 1)
            _drain_scatter(g0, 0)
            _fire(g0 + 2, 0)
            _drain_scatter(g0 + 1, 1)
            return 0

        lax.fori_loop(0, NG // 2 - 1, _grp2, 0)
        _fire(NG - 1, 1)
        _drain_scatter(NG - 2, 0)
        _drain_scatter(NG - 1, 1)
        plsc.subcore_barrier()
        pltpu.sync_copy(acc.at[pl.ds(s * RPTH, RPTH)],
                        out_hbm.at[(c * NP + p) * NS + s])
        plsc.subcore_barrier()


@functools.cache
def _sc_kernels():
    mesh = plsc.VectorSubcoreMesh(core_axis_name="c", subcore_axis_name="s",
                                  num_cores=NC, num_subcores=NS)
    deg = pl.kernel(
        _deg_body,
        out_type=jax.ShapeDtypeStruct((2 * NRD,), jnp.float32),
        mesh=mesh,
        scratch_types=[
            pltpu.VMEM((CPT, CHUNK), jnp.int32),
            pltpu.VMEM((CHUNK,), jnp.float32),
            pltpu.VMEM((RPTD,), jnp.float32),
            pltpu.VMEM_SHARED((NRD,), jnp.float32),
        ],
    )
    agg = pl.kernel(
        _agg_body,
        out_type=jax.ShapeDtypeStruct((NC * NP * NS, RPTH, HALF), jnp.float32),
        mesh=mesh,
        scratch_types=[
            pltpu.VMEM((CPT, CHUNK), jnp.int32),
            pltpu.VMEM((CPT, CHUNK), jnp.int32),
            pltpu.VMEM((CHUNK, HALF), jnp.float32),
            pltpu.VMEM((CHUNK, HALF), jnp.float32),
            pltpu.VMEM((CHUNK, HALF), jnp.float32),
            pltpu.VMEM((CHUNK, HALF), jnp.float32),
            pltpu.VMEM_SHARED((NROWH, HALF), jnp.float32),
            pltpu.SemaphoreType.DMA,
            pltpu.SemaphoreType.DMA,
        ],
    )
    return deg, agg


def _tcnorm_body(d_ref, ns_ref, nd_ref):
    a = lax.rsqrt(jnp.clip(d_ref[...], 1.0, None))
    ns_ref[...] = a[0:N].reshape(N, 1)
    nd_ref[...] = a[NRD:NRD + N].reshape(N, 1)


def _tcnorm(degs):
    return pl.pallas_call(
        _tcnorm_body,
        out_shape=[
            jax.ShapeDtypeStruct((N, 1), jnp.float32),
            jax.ShapeDtypeStruct((N, 1), jnp.float32),
        ],
    )(degs)


def _tc1_body(ns_ref, x_ref, w_ref, y_ref):
    h = x_ref[...] * ns_ref[...]
    y_ref[...] = jnp.dot(h, w_ref[...], precision=lax.Precision.HIGHEST,
                         preferred_element_type=jnp.float32)


def _tc1(ns, x, w1):
    nb = N // BN
    return pl.pallas_call(
        _tc1_body,
        grid=(NC, nb),
        in_specs=[
            pl.BlockSpec((BN, 1), lambda c, i: (i, 0)),
            pl.BlockSpec((BN, D), lambda c, i: (i, 0)),
            pl.BlockSpec((D, HALF), lambda c, i: (0, c)),
        ],
        out_specs=pl.BlockSpec((BN, HALF), lambda c, i: (c * nb + i, 0)),
        out_shape=jax.ShapeDtypeStruct((NC * N, HALF), jnp.float32),
    )(ns, x, w1)


def _tcmid_body(a0, a1, ns_ref, nd_ref, b_ref, w_ref, y_ref):
    h = jnp.concatenate([a0[0], a1[0]], axis=1)
    h = jax.nn.relu(h * nd_ref[...] + b_ref[...][None, :])
    y_ref[...] = jnp.dot(h * ns_ref[...], w_ref[...],
                         precision=lax.Precision.HIGHEST,
                         preferred_element_type=jnp.float32)


def _tcmid(agg, ns, nd, b, w):
    return pl.pallas_call(
        _tcmid_body,
        grid=(NC, N // BN),
        in_specs=[
            pl.BlockSpec((1, BN, HALF), lambda c, i: (0, i, 0)),
            pl.BlockSpec((1, BN, HALF), lambda c, i: (1, i, 0)),
            pl.BlockSpec((BN, 1), lambda c, i: (i, 0)),
            pl.BlockSpec((BN, 1), lambda c, i: (i, 0)),
            pl.BlockSpec((D,), lambda c, i: (0,)),
            pl.BlockSpec((D, HALF), lambda c, i: (0, c)),
        ],
        out_specs=pl.BlockSpec((BN, HALF), lambda c, i: (c * (N // BN) + i, 0)),
        out_shape=jax.ShapeDtypeStruct((NC * N, HALF), jnp.float32),
    )(agg, agg, ns, nd, b, w)


def _tcfin_body(a0, a1, nd_ref, b_ref, o_ref):
    h = jnp.concatenate([a0[0], a1[0]], axis=1)
    o_ref[...] = jax.nn.relu(h * nd_ref[...] + b_ref[...][None, :])


def _tcfin(agg, nd, b):
    return pl.pallas_call(
        _tcfin_body,
        grid=(N // BN,),
        in_specs=[
            pl.BlockSpec((1, BN, HALF), lambda i: (0, i, 0)),
            pl.BlockSpec((1, BN, HALF), lambda i: (1, i, 0)),
            pl.BlockSpec((BN, 1), lambda i: (i, 0)),
            pl.BlockSpec((D,), lambda i: (0,)),
        ],
        out_specs=pl.BlockSpec((BN, D), lambda i: (i, 0)),
        out_shape=jax.ShapeDtypeStruct((N, D), jnp.float32),
    )(agg, agg, nd, b)


def kernel(x, edge_index, W1, b1, W2, b2, W3, b3):
    src = edge_index[0].astype(jnp.int32)
    dst = edge_index[1].astype(jnp.int32)
    pad = EPAD - E
    padN = jnp.full((pad,), N, jnp.int32)
    src_g = jnp.concatenate([src, jnp.zeros((pad,), jnp.int32)])
    dst_p = jnp.concatenate([dst, padN])
    src_d = jnp.concatenate([src, padN])

    degidx = jnp.concatenate([src_d.reshape(NS, CPT, CHUNK),
                              dst_p.reshape(NS, CPT, CHUNK)])
    srcr = src_g.reshape(NS, CPT, CHUNK)
    srcw = jnp.concatenate([srcr, srcr + N])
    dstp = jnp.concatenate([
        jnp.where((dst_p >= p * NH) & (dst_p < (p + 1) * NH),
                  dst_p - p * NH, NH).reshape(NS, CPT, CHUNK)
        for p in range(NP)
    ])
    zer_agg = jnp.zeros((NS, RPTH, HALF), jnp.float32)

    deg_k, agg_k = _sc_kernels()

    def agg_kernel(y):
        flat = agg_k(y, srcw, dstp, zer_agg)
        a = flat.reshape(NC, NP, NS * RPTH, HALF)[:, :, :NH, :]
        return a.reshape(NC, N, HALF)

    degs = deg_k(degidx, jnp.ones((CHUNK,), jnp.float32),
                 jnp.zeros((NRD,), jnp.float32))
    ns, nd = _tcnorm(degs)
    y1 = _tc1(ns, x, W1)
    agg1 = agg_kernel(y1)
    y2 = _tcmid(agg1, ns, nd, b1, W2)
    agg2 = agg_kernel(y2)
    y3 = _tcmid(agg2, ns, nd, b2, W3)
    agg3 = agg_kernel(y3)
    return _tcfin(agg3, nd, b3)
